# Initial kernel scaffold; baseline (speedup 1.0000x reference)
import jax
import jax.numpy as jnp
from jax.experimental import pallas as pl


def kernel(x, edge_index, batch, num_graphs, W1_0, b1_0, W2_0, b2_0, W1_1, b1_1, W2_1, b2_1, W1_2, b1_2, W2_2, b2_2):
    raise NotImplementedError("write your pallas kernel here")


# speedup vs baseline: 4.6735x; 4.6735x over previous
import functools
import math

import jax
import jax.numpy as jnp
from jax import lax
from jax.experimental import pallas as pl
from jax.experimental.pallas import tpu as pltpu
from jax.experimental.pallas import tpu_sc as plsc

N = 10000
E = 320000
D = 128
G = 128
INV_BN = 1.0 / math.sqrt(1.0 + 1e-5)

NC, NS = 2, 16
NW = NC * NS
E_PER_TILE = E // NW
CH = 80
N_CHUNKS = E_PER_TILE // CH
N_PAD = 10240
ROWS_PER_TILE = N_PAD // NS
ZR = 128


def _sc_scatter(h, src, dst):
    mesh = plsc.VectorSubcoreMesh(core_axis_name="c", subcore_axis_name="s")

    @functools.partial(
        pl.kernel,
        out_type=jax.ShapeDtypeStruct((NC, N_PAD, D), jnp.float32),
        mesh=mesh,
        scratch_types=[
            pltpu.VMEM((CH,), jnp.int32),
            pltpu.VMEM((CH,), jnp.int32),
            pltpu.VMEM((CH, D), jnp.float32),
            pltpu.VMEM((ZR, D), jnp.float32),
            pltpu.VMEM_SHARED((N_PAD, D), jnp.float32),
            pltpu.SemaphoreType.DMA,
        ],
    )
    def k(h_hbm, src_hbm, dst_hbm, out_hbm, src_v, dst_v, rows_v, zero_v,
          acc_sh, sem):
        c = lax.axis_index("c")
        s = lax.axis_index("s")
        wid = s * NC + c

        zvec = jnp.zeros((16,), jnp.float32)

        def zrow(i, carry):
            for j in range(D // 16):
                zero_v[i, pl.ds(j * 16, 16)] = zvec
            return carry

        lax.fori_loop(0, ZR, zrow, 0)
        row0 = s * ROWS_PER_TILE
        for r in range(ROWS_PER_TILE // ZR):
            pltpu.sync_copy(zero_v, acc_sh.at[pl.ds(row0 + r * ZR, ZR)])
        plsc.subcore_barrier()

        ebase = wid * E_PER_TILE

        def body(i, carry):
            off = pl.multiple_of(ebase + i * CH, 8)
            pltpu.sync_copy(src_hbm.at[pl.ds(off, CH)], src_v)
            pltpu.async_copy(h_hbm.at[src_v], rows_v, sem).wait()
            pltpu.sync_copy(dst_hbm.at[pl.ds(off, CH)], dst_v)
            pltpu.sync_copy(rows_v, acc_sh.at[dst_v], add=True)
            return carry

        lax.fori_loop(0, N_CHUNKS, body, 0)
        plsc.subcore_barrier()

        pltpu.sync_copy(acc_sh.at[pl.ds(row0, ROWS_PER_TILE)],
                        out_hbm.at[c, pl.ds(row0, ROWS_PER_TILE)])

    return k(h, src, dst)


RB = 2000
N_RB = N // RB


def _tc_layer(h, a0, a1, batch3d, W1, b1, W2, b2):
    def body(h_ref, a0_ref, a1_ref, b_ref, W1_ref, b1_ref, W2_ref, b2_ref,
             o_ref, p_ref):
        i = pl.program_id(0)

        @pl.when(i == 0)
        def _():
            p_ref[...] = jnp.zeros_like(p_ref)

        m = a0_ref[...] + a1_ref[...] + h_ref[...]
        z = jnp.maximum(
            jnp.dot(m, W1_ref[...], preferred_element_type=jnp.float32)
            + b1_ref[...], 0.0)
        o = jnp.maximum(
            jnp.dot(z, W2_ref[...], preferred_element_type=jnp.float32)
            + b2_ref[...], 0.0) * INV_BN
        o_ref[...] = o
        sel = (lax.broadcasted_iota(jnp.int32, (G, RB), 0)
               == b_ref[...].reshape(1, RB)).astype(jnp.float32)
        p_ref[...] += jnp.dot(sel, o, preferred_element_type=jnp.float32)

    return pl.pallas_call(
        body,
        grid=(N_RB,),
        in_specs=[
            pl.BlockSpec((RB, D), lambda i: (i, 0)),
            pl.BlockSpec((RB, D), lambda i: (i, 0)),
            pl.BlockSpec((RB, D), lambda i: (i, 0)),
            pl.BlockSpec((1, 1, RB), lambda i: (i, 0, 0)),
            pl.BlockSpec((D, D), lambda i: (0, 0)),
            pl.BlockSpec((1, D), lambda i: (0, 0)),
            pl.BlockSpec((D, D), lambda i: (0, 0)),
            pl.BlockSpec((1, D), lambda i: (0, 0)),
        ],
        out_specs=[
            pl.BlockSpec((RB, D), lambda i: (i, 0)),
            pl.BlockSpec((G, D), lambda i: (0, 0)),
        ],
        out_shape=[
            jax.ShapeDtypeStruct((N, D), jnp.float32),
            jax.ShapeDtypeStruct((G, D), jnp.float32),
        ],
    )(h, a0, a1, batch3d, W1, b1, W2, b2)


def kernel(x, edge_index, batch, num_graphs, W1_0, b1_0, W2_0, b2_0,
           W1_1, b1_1, W2_1, b2_1, W1_2, b1_2, W2_2, b2_2):
    src = edge_index[0]
    dst = edge_index[1]
    batch3d = batch.reshape(N_RB, 1, RB)
    params = [(W1_0, b1_0, W2_0, b2_0), (W1_1, b1_1, W2_1, b2_1),
              (W1_2, b1_2, W2_2, b2_2)]
    h = x
    pooled = []
    for (W1, b1, W2, b2) in params:
        agg = _sc_scatter(h, src, dst)
        h, p = _tc_layer(h, agg[0, :N], agg[1, :N], batch3d,
                         W1, b1.reshape(1, D), W2, b2.reshape(1, D))
        pooled.append(p)
    return jnp.concatenate(pooled, axis=1)


# speedup vs baseline: 10.7824x; 2.3071x over previous
import functools
import math

import jax
import jax.numpy as jnp
from jax import lax
from jax.experimental import pallas as pl
from jax.experimental.pallas import tpu as pltpu
from jax.experimental.pallas import tpu_sc as plsc

N = 10000
E = 320000
D = 128
G = 128
INV_BN = 1.0 / math.sqrt(1.0 + 1e-5)

NC, NS = 2, 16
NW = NC * NS
E_PER_TILE = E // NW
CH = 80
N_CHUNKS = E_PER_TILE // CH
N_PAD = 10240
ROWS_PER_TILE = N_PAD // NS


def _sc_scatter(h, src2, dst3):
    mesh = plsc.VectorSubcoreMesh(core_axis_name="c", subcore_axis_name="s")

    @functools.partial(
        pl.kernel,
        out_type=jax.ShapeDtypeStruct((NC, N_PAD, D), jnp.float32),
        mesh=mesh,
        scratch_types=[
            pltpu.VMEM((E_PER_TILE,), jnp.int32),
            pltpu.VMEM((N_CHUNKS, CH), jnp.int32),
            pltpu.VMEM((CH, D), jnp.float32),
            pltpu.VMEM((CH, D), jnp.float32),
            pltpu.VMEM_SHARED((N_PAD, D), jnp.float32),
            pltpu.SemaphoreType.DMA,
            pltpu.SemaphoreType.DMA,
        ],
    )
    def k(h_hbm, src_hbm, dst_hbm, out_hbm, src_v, dst_v, rows_a, rows_b,
          acc_sh, sem_a, sem_b):
        c = lax.axis_index("c")
        s = lax.axis_index("s")
        wid = s * NC + c

        pltpu.sync_copy(src_hbm.at[wid], src_v)
        pltpu.sync_copy(dst_hbm.at[wid], dst_v)

        def fire(chunk, rows, sem):
            off = pl.multiple_of(chunk * CH, 8)
            return pltpu.async_copy(h_hbm.at[src_v.at[pl.ds(off, CH)]],
                                    rows, sem)

        def drain(rows, sem):
            pltpu.make_async_copy(h_hbm.at[src_v.at[pl.ds(0, CH)]],
                                  rows, sem).wait()

        def scat(chunk, rows):
            pltpu.sync_copy(rows, acc_sh.at[dst_v.at[chunk]], add=True)

        zvec = jnp.zeros((16,), jnp.float32)

        def zrow(i, carry):
            for j in range(D // 16):
                rows_a[i, pl.ds(j * 16, 16)] = zvec
            return carry

        lax.fori_loop(0, CH, zrow, 0)
        row0 = s * ROWS_PER_TILE
        for r in range(ROWS_PER_TILE // CH):
            pltpu.sync_copy(rows_a, acc_sh.at[pl.ds(row0 + r * CH, CH)])

        fire(0, rows_a, sem_a)
        fire(1, rows_b, sem_b)
        plsc.subcore_barrier()

        def body(kk, carry):
            drain(rows_a, sem_a)
            scat(2 * kk, rows_a)
            fire(2 * kk + 2, rows_a, sem_a)
            drain(rows_b, sem_b)
            scat(2 * kk + 1, rows_b)
            fire(2 * kk + 3, rows_b, sem_b)
            return carry

        lax.fori_loop(0, (N_CHUNKS - 3) // 2, body, 0)
        drain(rows_a, sem_a)
        scat(N_CHUNKS - 3, rows_a)
        fire(N_CHUNKS - 1, rows_a, sem_a)
        drain(rows_b, sem_b)
        scat(N_CHUNKS - 2, rows_b)
        drain(rows_a, sem_a)
        scat(N_CHUNKS - 1, rows_a)
        plsc.subcore_barrier()

        pltpu.sync_copy(acc_sh.at[pl.ds(row0, ROWS_PER_TILE)],
                        out_hbm.at[c, pl.ds(row0, ROWS_PER_TILE)])

    return k(h, src2, dst3)


RB = 2000
N_RB = N // RB


def _tc_layer(h, a0, a1, batch3d, W1, b1, W2, b2):
    def body(h_ref, a0_ref, a1_ref, b_ref, W1_ref, b1_ref, W2_ref, b2_ref,
             o_ref, p_ref):
        i = pl.program_id(0)

        @pl.when(i == 0)
        def _():
            p_ref[...] = jnp.zeros_like(p_ref)

        m = a0_ref[...] + a1_ref[...] + h_ref[...]
        z = jnp.maximum(
            jnp.dot(m, W1_ref[...], preferred_element_type=jnp.float32)
            + b1_ref[...], 0.0)
        o = jnp.maximum(
            jnp.dot(z, W2_ref[...], preferred_element_type=jnp.float32)
            + b2_ref[...], 0.0) * INV_BN
        o_ref[...] = o
        sel = (lax.broadcasted_iota(jnp.int32, (G, RB), 0)
               == b_ref[...].reshape(1, RB)).astype(jnp.float32)
        p_ref[...] += jnp.dot(sel, o, preferred_element_type=jnp.float32)

    return pl.pallas_call(
        body,
        grid=(N_RB,),
        in_specs=[
            pl.BlockSpec((RB, D), lambda i: (i, 0)),
            pl.BlockSpec((RB, D), lambda i: (i, 0)),
            pl.BlockSpec((RB, D), lambda i: (i, 0)),
            pl.BlockSpec((1, 1, RB), lambda i: (i, 0, 0)),
            pl.BlockSpec((D, D), lambda i: (0, 0)),
            pl.BlockSpec((1, D), lambda i: (0, 0)),
            pl.BlockSpec((D, D), lambda i: (0, 0)),
            pl.BlockSpec((1, D), lambda i: (0, 0)),
        ],
        out_specs=[
            pl.BlockSpec((RB, D), lambda i: (i, 0)),
            pl.BlockSpec((G, D), lambda i: (0, 0)),
        ],
        out_shape=[
            jax.ShapeDtypeStruct((N, D), jnp.float32),
            jax.ShapeDtypeStruct((G, D), jnp.float32),
        ],
    )(h, a0, a1, batch3d, W1, b1, W2, b2)


def kernel(x, edge_index, batch, num_graphs, W1_0, b1_0, W2_0, b2_0,
           W1_1, b1_1, W2_1, b2_1, W1_2, b1_2, W2_2, b2_2):
    src2 = edge_index[0].reshape(NW, E_PER_TILE)
    dst3 = edge_index[1].reshape(NW, N_CHUNKS, CH)
    batch3d = batch.reshape(N_RB, 1, RB)
    params = [(W1_0, b1_0, W2_0, b2_0), (W1_1, b1_1, W2_1, b2_1),
              (W1_2, b1_2, W2_2, b2_2)]
    h = x
    pooled = []
    for (W1, b1, W2, b2) in params:
        agg = _sc_scatter(h, src2, dst3)
        h, p = _tc_layer(h, agg[0, :N], agg[1, :N], batch3d,
                         W1, b1.reshape(1, D), W2, b2.reshape(1, D))
        pooled.append(p)
    return jnp.concatenate(pooled, axis=1)


# speedup vs baseline: 11.8551x; 1.0995x over previous
import functools
import math

import jax
import jax.numpy as jnp
from jax import lax
from jax.experimental import pallas as pl
from jax.experimental.pallas import tpu as pltpu
from jax.experimental.pallas import tpu_sc as plsc

N = 10000
E = 320000
D = 128
G = 128
INV_BN = 1.0 / math.sqrt(1.0 + 1e-5)

NC, NS = 2, 16
NW = NC * NS
E_PER_TILE = E // NW
CH = 128
N_CHUNKS = E_PER_TILE // CH
TAIL = E_PER_TILE - N_CHUNKS * CH
N_PAD = 10240
ROWS_PER_TILE = N_PAD // NS


def _sc_scatter(h, src2, dst1):
    mesh = plsc.VectorSubcoreMesh(core_axis_name="c", subcore_axis_name="s")

    @functools.partial(
        pl.kernel,
        out_type=jax.ShapeDtypeStruct((NC, N_PAD, D), jnp.float32),
        mesh=mesh,
        scratch_types=[
            pltpu.VMEM((E_PER_TILE,), jnp.int32),
            pltpu.VMEM((CH,), jnp.int32),
            pltpu.VMEM((CH,), jnp.int32),
            pltpu.VMEM((TAIL,), jnp.int32),
            pltpu.VMEM((CH, D), jnp.float32),
            pltpu.VMEM((CH, D), jnp.float32),
            pltpu.VMEM((TAIL, D), jnp.float32),
            pltpu.VMEM_SHARED((N_PAD, D), jnp.float32),
            pltpu.SemaphoreType.DMA,
            pltpu.SemaphoreType.DMA,
            pltpu.SemaphoreType.DMA,
            pltpu.SemaphoreType.DMA,
        ],
    )
    def k(h_hbm, src_hbm, dst_hbm, out_hbm, src_v, dst_a, dst_b, dst_t,
          rows_a, rows_b, rows_t, acc_sh, sem_a, sem_b, sem_da, sem_db):
        c = lax.axis_index("c")
        s = lax.axis_index("s")
        wid = s * NC + c
        ebase = wid * E_PER_TILE

        pltpu.sync_copy(src_hbm.at[wid], src_v)

        def fire(chunk, rows, sem, dstb, dsem):
            off = pl.multiple_of(chunk * CH, 8)
            pltpu.async_copy(dst_hbm.at[pl.ds(ebase + off, CH)], dstb, dsem)
            pltpu.async_copy(h_hbm.at[src_v.at[pl.ds(off, CH)]], rows, sem)

        def drain(rows, sem, dstb, dsem):
            pltpu.make_async_copy(dst_hbm.at[pl.ds(0, CH)], dstb, dsem).wait()
            pltpu.make_async_copy(h_hbm.at[src_v.at[pl.ds(0, CH)]],
                                  rows, sem).wait()

        def scat(rows, dstb):
            pltpu.sync_copy(rows, acc_sh.at[dstb], add=True)

        zvec = jnp.zeros((16,), jnp.float32)

        def zrow(i, carry):
            for j in range(D // 16):
                rows_a[i, pl.ds(j * 16, 16)] = zvec
            return carry

        lax.fori_loop(0, CH, zrow, 0)
        row0 = s * ROWS_PER_TILE
        for r in range(ROWS_PER_TILE // CH):
            pltpu.sync_copy(rows_a, acc_sh.at[pl.ds(row0 + r * CH, CH)])

        fire(0, rows_a, sem_a, dst_a, sem_da)
        fire(1, rows_b, sem_b, dst_b, sem_db)
        plsc.subcore_barrier()

        def body(kk, carry):
            drain(rows_a, sem_a, dst_a, sem_da)
            scat(rows_a, dst_a)
            fire(2 * kk + 2, rows_a, sem_a, dst_a, sem_da)
            drain(rows_b, sem_b, dst_b, sem_db)
            scat(rows_b, dst_b)
            fire(2 * kk + 3, rows_b, sem_b, dst_b, sem_db)
            return carry

        lax.fori_loop(0, (N_CHUNKS - 4) // 2, body, 0)
        drain(rows_a, sem_a, dst_a, sem_da)
        scat(rows_a, dst_a)
        fire(N_CHUNKS - 2, rows_a, sem_a, dst_a, sem_da)
        drain(rows_b, sem_b, dst_b, sem_db)
        scat(rows_b, dst_b)
        fire(N_CHUNKS - 1, rows_b, sem_b, dst_b, sem_db)
        drain(rows_a, sem_a, dst_a, sem_da)
        scat(rows_a, dst_a)
        drain(rows_b, sem_b, dst_b, sem_db)
        scat(rows_b, dst_b)
        toff = N_CHUNKS * CH
        pltpu.async_copy(dst_hbm.at[pl.ds(ebase + toff, TAIL)], dst_t, sem_da)
        pltpu.async_copy(h_hbm.at[src_v.at[pl.ds(toff, TAIL)]], rows_t, sem_a)
        pltpu.make_async_copy(dst_hbm.at[pl.ds(0, TAIL)], dst_t, sem_da).wait()
        pltpu.make_async_copy(h_hbm.at[src_v.at[pl.ds(0, TAIL)]],
                              rows_t, sem_a).wait()
        pltpu.sync_copy(rows_t, acc_sh.at[dst_t], add=True)
        plsc.subcore_barrier()

        pltpu.sync_copy(acc_sh.at[pl.ds(row0, ROWS_PER_TILE)],
                        out_hbm.at[c, pl.ds(row0, ROWS_PER_TILE)])

    return k(h, src2, dst1)


RB = 2000
N_RB = N // RB


def _tc_layer(h, a0, a1, batch3d, W1, b1, W2, b2):
    def body(h_ref, a0_ref, a1_ref, b_ref, W1_ref, b1_ref, W2_ref, b2_ref,
             o_ref, p_ref):
        i = pl.program_id(0)

        @pl.when(i == 0)
        def _():
            p_ref[...] = jnp.zeros_like(p_ref)

        m = a0_ref[...] + a1_ref[...] + h_ref[...]
        z = jnp.maximum(
            jnp.dot(m, W1_ref[...], preferred_element_type=jnp.float32)
            + b1_ref[...], 0.0)
        o = jnp.maximum(
            jnp.dot(z, W2_ref[...], preferred_element_type=jnp.float32)
            + b2_ref[...], 0.0) * INV_BN
        o_ref[...] = o
        sel = (lax.broadcasted_iota(jnp.int32, (G, RB), 0)
               == b_ref[...].reshape(1, RB)).astype(jnp.float32)
        p_ref[...] += jnp.dot(sel, o, preferred_element_type=jnp.float32)

    return pl.pallas_call(
        body,
        grid=(N_RB,),
        in_specs=[
            pl.BlockSpec((RB, D), lambda i: (i, 0)),
            pl.BlockSpec((RB, D), lambda i: (i, 0)),
            pl.BlockSpec((RB, D), lambda i: (i, 0)),
            pl.BlockSpec((1, 1, RB), lambda i: (i, 0, 0)),
            pl.BlockSpec((D, D), lambda i: (0, 0)),
            pl.BlockSpec((1, D), lambda i: (0, 0)),
            pl.BlockSpec((D, D), lambda i: (0, 0)),
            pl.BlockSpec((1, D), lambda i: (0, 0)),
        ],
        out_specs=[
            pl.BlockSpec((RB, D), lambda i: (i, 0)),
            pl.BlockSpec((G, D), lambda i: (0, 0)),
        ],
        out_shape=[
            jax.ShapeDtypeStruct((N, D), jnp.float32),
            jax.ShapeDtypeStruct((G, D), jnp.float32),
        ],
    )(h, a0, a1, batch3d, W1, b1, W2, b2)


def kernel(x, edge_index, batch, num_graphs, W1_0, b1_0, W2_0, b2_0,
           W1_1, b1_1, W2_1, b2_1, W1_2, b1_2, W2_2, b2_2):
    src2 = edge_index[0].reshape(NW, E_PER_TILE)
    dst1 = edge_index[1]
    batch3d = batch.reshape(N_RB, 1, RB)
    params = [(W1_0, b1_0, W2_0, b2_0), (W1_1, b1_1, W2_1, b2_1),
              (W1_2, b1_2, W2_2, b2_2)]
    h = x
    pooled = []
    for (W1, b1, W2, b2) in params:
        agg = _sc_scatter(h, src2, dst1)
        h, p = _tc_layer(h, agg[0, :N], agg[1, :N], batch3d,
                         W1, b1.reshape(1, D), W2, b2.reshape(1, D))
        pooled.append(p)
    return jnp.concatenate(pooled, axis=1)


# speedup vs baseline: 12.7665x; 1.0769x over previous
import functools
import math

import jax
import jax.numpy as jnp
from jax import lax
from jax.experimental import pallas as pl
from jax.experimental.pallas import tpu as pltpu
from jax.experimental.pallas import tpu_sc as plsc

N = 10000
E = 320000
D = 128
G = 128
INV_BN = 1.0 / math.sqrt(1.0 + 1e-5)

NC, NS = 2, 16
NW = NC * NS
E_PER_TILE = E // NW
CH = 128
N_CHUNKS = E_PER_TILE // CH
TAIL = E_PER_TILE - N_CHUNKS * CH
N_PAD = 10240
ROWS_PER_TILE = N_PAD // NS


def _sc_scatter(h, src2, dst1):
    mesh = plsc.VectorSubcoreMesh(core_axis_name="c", subcore_axis_name="s")

    @functools.partial(
        pl.kernel,
        out_type=jax.ShapeDtypeStruct((NC, N_PAD, D), jnp.float32),
        mesh=mesh,
        scratch_types=[
            pltpu.VMEM((E_PER_TILE,), jnp.int32),
            pltpu.VMEM((CH,), jnp.int32),
            pltpu.VMEM((CH,), jnp.int32),
            pltpu.VMEM((TAIL,), jnp.int32),
            pltpu.VMEM((CH, D), jnp.float32),
            pltpu.VMEM((CH, D), jnp.float32),
            pltpu.VMEM((TAIL, D), jnp.float32),
            pltpu.VMEM_SHARED((N_PAD, D), jnp.float32),
            pltpu.SemaphoreType.DMA,
            pltpu.SemaphoreType.DMA,
            pltpu.SemaphoreType.DMA,
            pltpu.SemaphoreType.DMA,
        ],
    )
    def k(h_hbm, src_hbm, dst_hbm, out_hbm, src_v, dst_a, dst_b, dst_t,
          rows_a, rows_b, rows_t, acc_sh, sem_a, sem_b, sem_da, sem_db):
        c = lax.axis_index("c")
        s = lax.axis_index("s")
        wid = s * NC + c
        ebase = wid * E_PER_TILE

        stage = pltpu.make_async_copy(src_hbm.at[wid], src_v, sem_da)
        stage.start()

        def fire(chunk, rows, sem, dstb, dsem):
            off = pl.multiple_of(chunk * CH, 8)
            pltpu.async_copy(dst_hbm.at[pl.ds(ebase + off, CH)], dstb, dsem)
            pltpu.async_copy(h_hbm.at[src_v.at[pl.ds(off, CH)]], rows, sem)

        def drain(rows, sem, dstb, dsem):
            pltpu.make_async_copy(dst_hbm.at[pl.ds(0, CH)], dstb, dsem).wait()
            pltpu.make_async_copy(h_hbm.at[src_v.at[pl.ds(0, CH)]],
                                  rows, sem).wait()

        def scat(rows, dstb):
            pltpu.sync_copy(rows, acc_sh.at[dstb], add=True)

        zvec = jnp.zeros((16,), jnp.float32)

        def zrow(i, carry):
            for j in range(D // 16):
                rows_b[i, pl.ds(j * 16, 16)] = zvec
            return carry

        lax.fori_loop(0, CH, zrow, 0)
        stage.wait()
        fire(0, rows_a, sem_a, dst_a, sem_da)
        row0 = s * ROWS_PER_TILE
        for r in range(ROWS_PER_TILE // CH):
            pltpu.sync_copy(rows_b, acc_sh.at[pl.ds(row0 + r * CH, CH)])
        fire(1, rows_b, sem_b, dst_b, sem_db)
        plsc.subcore_barrier()

        def body(kk, carry):
            drain(rows_a, sem_a, dst_a, sem_da)
            scat(rows_a, dst_a)
            fire(2 * kk + 2, rows_a, sem_a, dst_a, sem_da)
            drain(rows_b, sem_b, dst_b, sem_db)
            scat(rows_b, dst_b)
            fire(2 * kk + 3, rows_b, sem_b, dst_b, sem_db)
            return carry

        lax.fori_loop(0, (N_CHUNKS - 4) // 2, body, 0)
        drain(rows_a, sem_a, dst_a, sem_da)
        scat(rows_a, dst_a)
        fire(N_CHUNKS - 2, rows_a, sem_a, dst_a, sem_da)
        drain(rows_b, sem_b, dst_b, sem_db)
        scat(rows_b, dst_b)
        fire(N_CHUNKS - 1, rows_b, sem_b, dst_b, sem_db)
        drain(rows_a, sem_a, dst_a, sem_da)
        scat(rows_a, dst_a)
        drain(rows_b, sem_b, dst_b, sem_db)
        scat(rows_b, dst_b)
        toff = N_CHUNKS * CH
        pltpu.async_copy(dst_hbm.at[pl.ds(ebase + toff, TAIL)], dst_t, sem_da)
        pltpu.async_copy(h_hbm.at[src_v.at[pl.ds(toff, TAIL)]], rows_t, sem_a)
        pltpu.make_async_copy(dst_hbm.at[pl.ds(0, TAIL)], dst_t, sem_da).wait()
        pltpu.make_async_copy(h_hbm.at[src_v.at[pl.ds(0, TAIL)]],
                              rows_t, sem_a).wait()
        pltpu.sync_copy(rows_t, acc_sh.at[dst_t], add=True)
        plsc.subcore_barrier()

        pltpu.sync_copy(acc_sh.at[pl.ds(row0, ROWS_PER_TILE)],
                        out_hbm.at[c, pl.ds(row0, ROWS_PER_TILE)])

    return k(h, src2, dst1)


RB = 2000
N_RB = N // RB


def _tc_layer(h, agg, batch3d, W1, b1, W2, b2):
    def body(h_ref, a_ref, b_ref, W1_ref, b1_ref, W2_ref, b2_ref,
             o_ref, p_ref):
        i = pl.program_id(0)

        @pl.when(i == 0)
        def _():
            p_ref[...] = jnp.zeros_like(p_ref)

        m = a_ref[0] + a_ref[1] + h_ref[...]
        z = jnp.maximum(
            jnp.dot(m, W1_ref[...], preferred_element_type=jnp.float32)
            + b1_ref[...], 0.0)
        o = jnp.maximum(
            jnp.dot(z, W2_ref[...], preferred_element_type=jnp.float32)
            + b2_ref[...], 0.0) * INV_BN
        o_ref[...] = o
        sel = (lax.broadcasted_iota(jnp.int32, (G, RB), 0)
               == b_ref[...].reshape(1, RB)).astype(jnp.float32)
        p_ref[...] += jnp.dot(sel, o, preferred_element_type=jnp.float32)

    return pl.pallas_call(
        body,
        grid=(N_RB,),
        in_specs=[
            pl.BlockSpec((RB, D), lambda i: (i, 0)),
            pl.BlockSpec((2, RB, D), lambda i: (0, i, 0)),
            pl.BlockSpec((1, 1, RB), lambda i: (i, 0, 0)),
            pl.BlockSpec((D, D), lambda i: (0, 0)),
            pl.BlockSpec((1, D), lambda i: (0, 0)),
            pl.BlockSpec((D, D), lambda i: (0, 0)),
            pl.BlockSpec((1, D), lambda i: (0, 0)),
        ],
        out_specs=[
            pl.BlockSpec((RB, D), lambda i: (i, 0)),
            pl.BlockSpec((G, D), lambda i: (0, 0)),
        ],
        out_shape=[
            jax.ShapeDtypeStruct((N, D), jnp.float32),
            jax.ShapeDtypeStruct((G, D), jnp.float32),
        ],
    )(h, agg, batch3d, W1, b1, W2, b2)


def kernel(x, edge_index, batch, num_graphs, W1_0, b1_0, W2_0, b2_0,
           W1_1, b1_1, W2_1, b2_1, W1_2, b1_2, W2_2, b2_2):
    src2 = edge_index[0].reshape(NW, E_PER_TILE)
    dst1 = edge_index[1]
    batch3d = batch.reshape(N_RB, 1, RB)
    params = [(W1_0, b1_0, W2_0, b2_0), (W1_1, b1_1, W2_1, b2_1),
              (W1_2, b1_2, W2_2, b2_2)]
    h = x
    pooled = []
    for (W1, b1, W2, b2) in params:
        agg = _sc_scatter(h, src2, dst1)
        h, p = _tc_layer(h, agg, batch3d,
                         W1, b1.reshape(1, D), W2, b2.reshape(1, D))
        pooled.append(p)
    return jnp.concatenate(pooled, axis=1)
